# baseline (speedup 1.0000x reference)
import functools

import jax
import jax.numpy as jnp
from jax import lax
from jax.experimental import pallas as pl
from jax.experimental.pallas import tpu as pltpu
from jax.experimental.pallas import tpu_sc as plsc

_NC = 2
_NS = 16
_NW = _NC * _NS
_L = 16
_IDX_CHUNK = 128


def _sc_scores(user_ids, item_ids, neg_item_ids, user_table, item_table):
    B = user_ids.shape[0]
    D = user_table.shape[1]
    bpw = B // _NW
    nchunk = bpw // _IDX_CHUNK

    uids3 = user_ids.reshape(_NW, nchunk, _IDX_CHUNK)
    pids3 = item_ids.reshape(_NW, nchunk, _IDX_CHUNK)
    nids3 = neg_item_ids.reshape(_NW, nchunk, _IDX_CHUNK)

    mesh = plsc.VectorSubcoreMesh(core_axis_name="c", subcore_axis_name="s")

    @functools.partial(
        pl.kernel,
        out_type=jax.ShapeDtypeStruct((B,), jnp.float32),
        mesh=mesh,
        compiler_params=pltpu.CompilerParams(
            needs_layout_passes=False, use_tc_tiling_on_sc=False),
        scratch_types=[
            pltpu.VMEM((nchunk, _IDX_CHUNK), jnp.int32),
            pltpu.VMEM((nchunk, _IDX_CHUNK), jnp.int32),
            pltpu.VMEM((nchunk, _IDX_CHUNK), jnp.int32),
            pltpu.VMEM((bpw, D), jnp.float32),
            pltpu.VMEM((bpw, D), jnp.float32),
            pltpu.VMEM((bpw, D), jnp.float32),
            pltpu.VMEM((bpw,), jnp.float32),
            pltpu.SemaphoreType.DMA,
        ],
    )
    def sc_kernel(uids_hbm, pids_hbm, nids_hbm, utab_hbm, itab_hbm, out_hbm,
                  uidx_v, pidx_v, nidx_v, u_v, p_v, n_v, t_v, sem):
        wid = lax.axis_index("s") * _NC + lax.axis_index("c")

        pltpu.sync_copy(uids_hbm.at[wid], uidx_v)
        pltpu.sync_copy(pids_hbm.at[wid], pidx_v)
        pltpu.sync_copy(nids_hbm.at[wid], nidx_v)

        copies = []
        for c in range(nchunk):
            rows_sl = pl.ds(c * _IDX_CHUNK, _IDX_CHUNK)
            copies.append(pltpu.async_copy(
                utab_hbm.at[uidx_v.at[c]], u_v.at[rows_sl], sem))
            copies.append(pltpu.async_copy(
                itab_hbm.at[pidx_v.at[c]], p_v.at[rows_sl], sem))
            copies.append(pltpu.async_copy(
                itab_hbm.at[nidx_v.at[c]], n_v.at[rows_sl], sem))
        for cp in copies:
            cp.wait()

        iota = lax.iota(jnp.int32, _L)

        def body(g, carry):
            rows = g * _L + iota
            acc = jnp.zeros((_L,), jnp.float32)
            for d in range(D):
                dcol = jnp.full((_L,), d, jnp.int32)
                uu = plsc.load_gather(u_v, [rows, dcol])
                pp = plsc.load_gather(p_v, [rows, dcol])
                nn = plsc.load_gather(n_v, [rows, dcol])
                acc = acc + uu * (pp - nn)
            t_v[pl.ds(g * _L, _L)] = acc
            return carry

        lax.fori_loop(0, bpw // _L, body, 0)
        pltpu.sync_copy(t_v, out_hbm.at[pl.ds(wid * bpw, bpw)])

    return sc_kernel(uids3, pids3, nids3, user_table, item_table)


def _tc_loss_body(x_ref, o_ref):
    x = x_ref[...]
    ls = jnp.minimum(x, 0.0) - jnp.log1p(jnp.exp(-jnp.abs(x)))
    o_ref[...] = jnp.broadcast_to(-jnp.mean(ls), (1, 1))


def kernel(user_ids, item_ids, neg_item_ids, user_table, item_table):
    scores = _sc_scores(user_ids, item_ids, neg_item_ids,
                        user_table, item_table)
    B = scores.shape[0]
    loss2d = pl.pallas_call(
        _tc_loss_body,
        out_shape=jax.ShapeDtypeStruct((1, 1), jnp.float32),
    )(scores.reshape(128, B // 128))
    return loss2d[0, 0]


# speedup vs baseline: 2.4480x; 2.4480x over previous
import functools

import jax
import jax.numpy as jnp
from jax import lax
from jax.experimental import pallas as pl
from jax.experimental.pallas import tpu as pltpu
from jax.experimental.pallas import tpu_sc as plsc

_NC = 2
_NS = 16
_NW = _NC * _NS
_L = 16
_TB = 8


def _sc_scores(user_ids, item_ids, neg_item_ids, user_table, item_table):
    B = user_ids.shape[0]
    V, D = user_table.shape
    bpw = B // _NW
    ngroup = bpw // _L

    uids2 = user_ids.reshape(_NW, bpw)
    pids2 = item_ids.reshape(_NW, bpw)
    nids2 = neg_item_ids.reshape(_NW, bpw)
    utab3 = user_table.reshape(V // _TB, _TB, D)
    itab3 = item_table.reshape(V // _TB, _TB, D)

    mesh = plsc.VectorSubcoreMesh(core_axis_name="c", subcore_axis_name="s")

    @functools.partial(
        pl.kernel,
        out_type=jax.ShapeDtypeStruct((B,), jnp.float32),
        mesh=mesh,
        compiler_params=pltpu.CompilerParams(needs_layout_passes=False),
        scratch_types=[
            pltpu.VMEM((bpw,), jnp.int32),
            pltpu.VMEM((bpw,), jnp.int32),
            pltpu.VMEM((bpw,), jnp.int32),
            pltpu.VMEM((bpw // 2, D), jnp.float32),
            pltpu.VMEM((bpw // 2, D), jnp.float32),
            pltpu.VMEM((bpw // 2, D), jnp.float32),
            pltpu.VMEM((bpw,), jnp.float32),
            pltpu.SemaphoreType.DMA,
            pltpu.SemaphoreType.DMA,
            pltpu.SemaphoreType.DMA,
        ],
    )
    def sc_kernel(uids_hbm, pids_hbm, nids_hbm, utab_hbm, itab_hbm, out_hbm,
                  uidx_v, pidx_v, nidx_v, u_v, p_v, n_v, t_v,
                  sem_u, sem_p, sem_n):
        wid = lax.axis_index("s") * _NC + lax.axis_index("c")

        pltpu.sync_copy(uids_hbm.at[wid], uidx_v)
        pltpu.sync_copy(pids_hbm.at[wid], pidx_v)
        pltpu.sync_copy(nids_hbm.at[wid], nidx_v)

        iota = lax.iota(jnp.int32, _L)

        for h in range(2):
            hbase = h * (ngroup // 2)

            def fire_body(g, carry, hbase=hbase):
                sl = pl.ds((hbase + g) * _L, _L)
                uu = uidx_v[sl]
                pp = pidx_v[sl]
                nn = nidx_v[sl]
                for j in range(_L):
                    r = g * _L + j
                    pltpu.async_copy(
                        utab_hbm.at[uu[j] >> 3, uu[j] & (_TB - 1)],
                        u_v.at[r], sem_u)
                    pltpu.async_copy(
                        itab_hbm.at[pp[j] >> 3, pp[j] & (_TB - 1)],
                        p_v.at[r], sem_p)
                    pltpu.async_copy(
                        itab_hbm.at[nn[j] >> 3, nn[j] & (_TB - 1)],
                        n_v.at[r], sem_n)
                return carry

            lax.fori_loop(0, ngroup // 2, fire_body, 0)

            def drain_body(g, carry):
                for j in range(_L):
                    r = g * _L + j
                    pltpu.make_async_copy(utab_hbm.at[0, 0],
                                          u_v.at[r], sem_u).wait()
                    pltpu.make_async_copy(itab_hbm.at[0, 0],
                                          p_v.at[r], sem_p).wait()
                    pltpu.make_async_copy(itab_hbm.at[0, 0],
                                          n_v.at[r], sem_n).wait()
                return carry

            lax.fori_loop(0, ngroup // 2, drain_body, 0)

            def body(g, carry, hbase=hbase):
                rows = g * _L + iota
                acc = jnp.zeros((_L,), jnp.float32)
                for d in range(D):
                    dcol = jnp.full((_L,), d, jnp.int32)
                    uu = plsc.load_gather(u_v, [rows, dcol])
                    pp = plsc.load_gather(p_v, [rows, dcol])
                    nn = plsc.load_gather(n_v, [rows, dcol])
                    acc = acc + uu * (pp - nn)
                t_v[pl.ds((hbase + g) * _L, _L)] = acc
                return carry

            lax.fori_loop(0, ngroup // 2, body, 0)

        pltpu.sync_copy(t_v, out_hbm.at[pl.ds(wid * bpw, bpw)])

    return sc_kernel(uids2, pids2, nids2, utab3, itab3)


def _tc_loss_body(x_ref, o_ref):
    x = x_ref[...]
    ls = jnp.minimum(x, 0.0) - jnp.log1p(jnp.exp(-jnp.abs(x)))
    o_ref[...] = jnp.broadcast_to(-jnp.mean(ls), (1, 1))


def kernel(user_ids, item_ids, neg_item_ids, user_table, item_table):
    scores = _sc_scores(user_ids, item_ids, neg_item_ids,
                        user_table, item_table)
    B = scores.shape[0]
    loss2d = pl.pallas_call(
        _tc_loss_body,
        out_shape=jax.ShapeDtypeStruct((1, 1), jnp.float32),
    )(scores.reshape(128, B // 128))
    return loss2d[0, 0]
